# Initial kernel scaffold; baseline (speedup 1.0000x reference)
import functools

import jax
import jax.numpy as jnp
from jax.experimental import pallas as pl
from jax.experimental.pallas import tpu as pltpu

N = 4096
BLK = 256
NEG = jnp.float32(-9e15)
ALPHA = jnp.float32(0.2)


def _prep_kernel(x_ref, w_ref, a_ref, h_ref, f1_ref, f2_ref, *, f):
    h = jnp.dot(x_ref[...], w_ref[...], preferred_element_type=jnp.float32)
    h_ref[...] = h
    f1_ref[...] = jnp.dot(h, a_ref[:f, :], preferred_element_type=jnp.float32)
    f2_ref[...] = jax.lax.dot_general(
        a_ref[f:, :], h, (((0,), (1,)), ((), ())),
        preferred_element_type=jnp.float32)


def _prep(x, w, a):
    f = w.shape[1]
    din = x.shape[1]
    return pl.pallas_call(
        functools.partial(_prep_kernel, f=f),
        grid=(1,),
        in_specs=[
            pl.BlockSpec((N, din), lambda i: (0, 0)),
            pl.BlockSpec((din, f), lambda i: (0, 0)),
            pl.BlockSpec((2 * f, 1), lambda i: (0, 0)),
        ],
        out_specs=[
            pl.BlockSpec((N, f), lambda i: (0, 0)),
            pl.BlockSpec((N, 1), lambda i: (0, 0)),
            pl.BlockSpec((1, N), lambda i: (0, 0)),
        ],
        out_shape=[
            jax.ShapeDtypeStruct((N, f), jnp.float32),
            jax.ShapeDtypeStruct((N, 1), jnp.float32),
            jax.ShapeDtypeStruct((1, N), jnp.float32),
        ],
    )(x, w, a)


def _attn_block(adj_ref, f1_ref, f2_ref):
    e = f1_ref[...] + f2_ref[...]
    e = jnp.where(e >= 0, e, e * ALPHA)
    e = jnp.where(adj_ref[...] > 0, e, NEG)
    m = jnp.max(e, axis=1, keepdims=True)
    p = jnp.exp(e - m)
    s = jnp.sum(p, axis=1, keepdims=True)
    return p / s


def _layer1_kernel(adj_ref, h_ref, f1_ref, f2_ref, out_ref):
    attn = _attn_block(adj_ref, f1_ref, f2_ref)
    hp = jnp.dot(attn, h_ref[...], preferred_element_type=jnp.float32)
    out_ref[...] = jnp.maximum(hp, 0.0)


def _layer2_kernel(adj_ref, h_ref, f1_ref, f2_ref, c_ref, b_ref,
                   attn_ref, logits_ref):
    attn = _attn_block(adj_ref, f1_ref, f2_ref)
    attn_ref[...] = attn
    hp = jnp.dot(attn, h_ref[...], preferred_element_type=jnp.float32)
    logits_ref[...] = (
        jnp.dot(hp, c_ref[...], preferred_element_type=jnp.float32)
        + b_ref[...])


def _layer1(adj, h, f1, f2):
    f = h.shape[1]
    return pl.pallas_call(
        _layer1_kernel,
        grid=(N // BLK,),
        in_specs=[
            pl.BlockSpec((BLK, N), lambda i: (i, 0)),
            pl.BlockSpec((N, f), lambda i: (0, 0)),
            pl.BlockSpec((BLK, 1), lambda i: (i, 0)),
            pl.BlockSpec((1, N), lambda i: (0, 0)),
        ],
        out_specs=pl.BlockSpec((BLK, f), lambda i: (i, 0)),
        out_shape=jax.ShapeDtypeStruct((N, f), jnp.float32),
        compiler_params=pltpu.CompilerParams(
            dimension_semantics=("parallel",)),
    )(adj, h, f1, f2)


def _layer2(adj, h, f1, f2, c, b):
    f = h.shape[1]
    out = c.shape[1]
    return pl.pallas_call(
        _layer2_kernel,
        grid=(N // BLK,),
        in_specs=[
            pl.BlockSpec((BLK, N), lambda i: (i, 0)),
            pl.BlockSpec((N, f), lambda i: (0, 0)),
            pl.BlockSpec((BLK, 1), lambda i: (i, 0)),
            pl.BlockSpec((1, N), lambda i: (0, 0)),
            pl.BlockSpec((f, out), lambda i: (0, 0)),
            pl.BlockSpec((1, out), lambda i: (0, 0)),
        ],
        out_specs=[
            pl.BlockSpec((BLK, N), lambda i: (i, 0)),
            pl.BlockSpec((BLK, out), lambda i: (i, 0)),
        ],
        out_shape=[
            jax.ShapeDtypeStruct((N, N), jnp.float32),
            jax.ShapeDtypeStruct((N, out), jnp.float32),
        ],
        compiler_params=pltpu.CompilerParams(
            dimension_semantics=("parallel",)),
    )(adj, h, f1, f2, c, b)


@jax.jit
def kernel(ft, adj, W1, a1, W2, a2, embd2class, bias):
    h, f1, f2 = _prep(ft, W1, a1)
    embd = _layer1(adj, h, f1, f2)
    g, g1, g2 = _prep(embd, W2, a2)
    attention, logits = _layer2(adj, g, g1, g2, embd2class, bias)
    return (logits, embd, attention)


# speedup vs baseline: 1.9384x; 1.9384x over previous
import functools

import jax
import jax.numpy as jnp
from jax.experimental import pallas as pl
from jax.experimental.pallas import tpu as pltpu

N = 4096
BLK = 256
NEG = -9e15
ALPHA = 0.2


def _prep_kernel(x_ref, w_ref, a_ref, h_ref, f1_ref, f2_ref, *, f):
    h = jnp.dot(x_ref[...], w_ref[...], preferred_element_type=jnp.float32)
    h_ref[...] = h
    f1_ref[...] = jnp.dot(h, a_ref[:f, :], preferred_element_type=jnp.float32)
    f2_ref[...] = jax.lax.dot_general(
        a_ref[f:, :], h, (((0,), (1,)), ((), ())),
        preferred_element_type=jnp.float32)


def _prep(x, w, a):
    f = w.shape[1]
    din = x.shape[1]
    return pl.pallas_call(
        functools.partial(_prep_kernel, f=f),
        grid=(1,),
        in_specs=[
            pl.BlockSpec((N, din), lambda i: (0, 0)),
            pl.BlockSpec((din, f), lambda i: (0, 0)),
            pl.BlockSpec((2 * f, 1), lambda i: (0, 0)),
        ],
        out_specs=[
            pl.BlockSpec((N, f), lambda i: (0, 0)),
            pl.BlockSpec((N, 1), lambda i: (0, 0)),
            pl.BlockSpec((1, N), lambda i: (0, 0)),
        ],
        out_shape=[
            jax.ShapeDtypeStruct((N, f), jnp.float32),
            jax.ShapeDtypeStruct((N, 1), jnp.float32),
            jax.ShapeDtypeStruct((1, N), jnp.float32),
        ],
    )(x, w, a)


def _attn_block(adj_ref, f1_ref, f2_ref):
    e = f1_ref[...] + f2_ref[...]
    e = jnp.where(e >= 0, e, e * jnp.float32(ALPHA))
    e = jnp.where(adj_ref[...] > 0, e, jnp.float32(NEG))
    m = jnp.max(e, axis=1, keepdims=True)
    p = jnp.exp(e - m)
    s = jnp.sum(p, axis=1, keepdims=True)
    return p / s


def _layer1_kernel(adj_ref, h_ref, f1_ref, f2_ref, out_ref):
    attn = _attn_block(adj_ref, f1_ref, f2_ref)
    hp = jnp.dot(attn, h_ref[...], preferred_element_type=jnp.float32)
    out_ref[...] = jnp.maximum(hp, 0.0)


def _layer2_kernel(adj_ref, h_ref, f1_ref, f2_ref, c_ref, b_ref,
                   attn_ref, logits_ref):
    attn = _attn_block(adj_ref, f1_ref, f2_ref)
    attn_ref[...] = attn
    hp = jnp.dot(attn, h_ref[...], preferred_element_type=jnp.float32)
    logits_ref[...] = (
        jnp.dot(hp, c_ref[...], preferred_element_type=jnp.float32)
        + b_ref[...])


def _layer1(adj, h, f1, f2):
    f = h.shape[1]
    return pl.pallas_call(
        _layer1_kernel,
        grid=(N // BLK,),
        in_specs=[
            pl.BlockSpec((BLK, N), lambda i: (i, 0)),
            pl.BlockSpec((N, f), lambda i: (0, 0)),
            pl.BlockSpec((BLK, 1), lambda i: (i, 0)),
            pl.BlockSpec((1, N), lambda i: (0, 0)),
        ],
        out_specs=pl.BlockSpec((BLK, f), lambda i: (i, 0)),
        out_shape=jax.ShapeDtypeStruct((N, f), jnp.float32),
        compiler_params=pltpu.CompilerParams(
            dimension_semantics=("parallel",)),
    )(adj, h, f1, f2)


def _layer2(adj, h, f1, f2, c, b):
    f = h.shape[1]
    out = c.shape[1]
    return pl.pallas_call(
        _layer2_kernel,
        grid=(N // BLK,),
        in_specs=[
            pl.BlockSpec((BLK, N), lambda i: (i, 0)),
            pl.BlockSpec((N, f), lambda i: (0, 0)),
            pl.BlockSpec((BLK, 1), lambda i: (i, 0)),
            pl.BlockSpec((1, N), lambda i: (0, 0)),
            pl.BlockSpec((f, out), lambda i: (0, 0)),
            pl.BlockSpec((1, out), lambda i: (0, 0)),
        ],
        out_specs=[
            pl.BlockSpec((BLK, N), lambda i: (i, 0)),
            pl.BlockSpec((BLK, out), lambda i: (i, 0)),
        ],
        out_shape=[
            jax.ShapeDtypeStruct((N, N), jnp.float32),
            jax.ShapeDtypeStruct((N, out), jnp.float32),
        ],
        compiler_params=pltpu.CompilerParams(
            dimension_semantics=("parallel",)),
    )(adj, h, f1, f2, c, b)


@jax.jit
def kernel(ft, adj, W1, a1, W2, a2, embd2class, bias):
    h, f1, f2 = _prep(ft, W1, a1)
    embd = _layer1(adj, h, f1, f2)
    g, g1, g2 = _prep(embd, W2, a2)
    attention, logits = _layer2(adj, g, g1, g2, embd2class, bias)
    return (logits, embd, attention)


# speedup vs baseline: 2.1005x; 1.0836x over previous
import functools

import jax
import jax.numpy as jnp
from jax.experimental import pallas as pl
from jax.experimental.pallas import tpu as pltpu

N = 4096
BLK = 256
ALPHA = 0.2


def _prep_kernel(x_ref, w_ref, a_ref, haug_ref, c_ref, d_ref, f2_ref,
                 g_ref, cs_ref, *, f):
    h = jnp.dot(x_ref[...], w_ref[...], preferred_element_type=jnp.float32)
    haug_ref[:, :f] = h
    haug_ref[:, f:] = jnp.ones((N, 1), jnp.float32)
    f1 = jnp.dot(h, a_ref[:f, :], preferred_element_type=jnp.float32)
    f2 = jax.lax.dot_general(
        a_ref[f:, :], h, (((0,), (1,)), ((), ())),
        preferred_element_type=jnp.float32)
    f2_ref[...] = f2
    g_ref[...] = f2 * jnp.float32(ALPHA)
    m2 = jnp.max(f2)
    t = f1 + m2
    m = jnp.maximum(t, t * jnp.float32(ALPHA))
    c_ref[...] = f1 - m
    d_ref[...] = f1 * jnp.float32(ALPHA) - m
    cs_ref[...] = jnp.sum(h, axis=0, keepdims=True)


def _prep(x, w, a):
    f = w.shape[1]
    din = x.shape[1]
    return pl.pallas_call(
        functools.partial(_prep_kernel, f=f),
        grid=(1,),
        in_specs=[
            pl.BlockSpec((N, din), lambda i: (0, 0)),
            pl.BlockSpec((din, f), lambda i: (0, 0)),
            pl.BlockSpec((2 * f, 1), lambda i: (0, 0)),
        ],
        out_specs=[
            pl.BlockSpec((N, f + 1), lambda i: (0, 0)),
            pl.BlockSpec((N, 1), lambda i: (0, 0)),
            pl.BlockSpec((N, 1), lambda i: (0, 0)),
            pl.BlockSpec((1, N), lambda i: (0, 0)),
            pl.BlockSpec((1, N), lambda i: (0, 0)),
            pl.BlockSpec((1, f), lambda i: (0, 0)),
        ],
        out_shape=[
            jax.ShapeDtypeStruct((N, f + 1), jnp.float32),
            jax.ShapeDtypeStruct((N, 1), jnp.float32),
            jax.ShapeDtypeStruct((N, 1), jnp.float32),
            jax.ShapeDtypeStruct((1, N), jnp.float32),
            jax.ShapeDtypeStruct((1, N), jnp.float32),
            jax.ShapeDtypeStruct((1, f), jnp.float32),
        ],
    )(x, w, a)


def _softmax_block(adj_ref, haug_ref, c_ref, d_ref, f2_ref, g_ref, cs_ref, f):
    q = jnp.maximum(c_ref[...] + f2_ref[...], d_ref[...] + g_ref[...])
    pm = jnp.exp(q) * adj_ref[...]
    hpz = jnp.dot(pm, haug_ref[...], preferred_element_type=jnp.float32)
    s = hpz[:, f:]
    z = jnp.where(s == 0, jnp.float32(1.0), jnp.float32(0.0))
    r = jnp.float32(1.0) / (s + z * jnp.float32(N))
    hp = (hpz[:, :f] + z * cs_ref[...]) * r
    return pm, z, r, hp


def _layer1_kernel(adj_ref, haug_ref, c_ref, d_ref, f2_ref, g_ref, cs_ref,
                   out_ref, *, f):
    _, _, _, hp = _softmax_block(adj_ref, haug_ref, c_ref, d_ref, f2_ref,
                                 g_ref, cs_ref, f)
    out_ref[...] = jnp.maximum(hp, 0.0)


def _layer2_kernel(adj_ref, haug_ref, c_ref, d_ref, f2_ref, g_ref, cs_ref,
                   e2c_ref, b_ref, attn_ref, logits_ref, *, f):
    pm, z, r, hp = _softmax_block(adj_ref, haug_ref, c_ref, d_ref, f2_ref,
                                  g_ref, cs_ref, f)
    attn_ref[...] = (pm + z) * r
    logits_ref[...] = (
        jnp.dot(hp, e2c_ref[...], preferred_element_type=jnp.float32)
        + b_ref[...])


def _common_specs(f):
    return [
        pl.BlockSpec((BLK, N), lambda i: (i, 0)),
        pl.BlockSpec((N, f + 1), lambda i: (0, 0)),
        pl.BlockSpec((BLK, 1), lambda i: (i, 0)),
        pl.BlockSpec((BLK, 1), lambda i: (i, 0)),
        pl.BlockSpec((1, N), lambda i: (0, 0)),
        pl.BlockSpec((1, N), lambda i: (0, 0)),
        pl.BlockSpec((1, f), lambda i: (0, 0)),
    ]


def _layer1(adj, haug, c, d, f2, g, cs):
    f = haug.shape[1] - 1
    return pl.pallas_call(
        functools.partial(_layer1_kernel, f=f),
        grid=(N // BLK,),
        in_specs=_common_specs(f),
        out_specs=pl.BlockSpec((BLK, f), lambda i: (i, 0)),
        out_shape=jax.ShapeDtypeStruct((N, f), jnp.float32),
        compiler_params=pltpu.CompilerParams(
            dimension_semantics=("parallel",)),
    )(adj, haug, c, d, f2, g, cs)


def _layer2(adj, haug, c, d, f2, g, cs, e2c, b):
    f = haug.shape[1] - 1
    out = e2c.shape[1]
    return pl.pallas_call(
        functools.partial(_layer2_kernel, f=f),
        grid=(N // BLK,),
        in_specs=_common_specs(f) + [
            pl.BlockSpec((f, out), lambda i: (0, 0)),
            pl.BlockSpec((1, out), lambda i: (0, 0)),
        ],
        out_specs=[
            pl.BlockSpec((BLK, N), lambda i: (i, 0)),
            pl.BlockSpec((BLK, out), lambda i: (i, 0)),
        ],
        out_shape=[
            jax.ShapeDtypeStruct((N, N), jnp.float32),
            jax.ShapeDtypeStruct((N, out), jnp.float32),
        ],
        compiler_params=pltpu.CompilerParams(
            dimension_semantics=("parallel",)),
    )(adj, haug, c, d, f2, g, cs, e2c, b)


@jax.jit
def kernel(ft, adj, W1, a1, W2, a2, embd2class, bias):
    haug, c, d, f2, g, cs = _prep(ft, W1, a1)
    embd = _layer1(adj, haug, c, d, f2, g, cs)
    haug2, c2, d2, f22, g2, cs2 = _prep(embd, W2, a2)
    attention, logits = _layer2(adj, haug2, c2, d2, f22, g2, cs2,
                                embd2class, bias)
    return (logits, embd, attention)


# speedup vs baseline: 2.1947x; 1.0448x over previous
import functools

import jax
import jax.numpy as jnp
from jax.experimental import pallas as pl
from jax.experimental.pallas import tpu as pltpu

N = 4096
BLK = 512
ALPHA = 0.2


def _prep_kernel(x_ref, w_ref, a_ref, haug_ref, c_ref, d_ref, f2_ref,
                 g_ref, cs_ref, *, f):
    h = jnp.dot(x_ref[...], w_ref[...], preferred_element_type=jnp.float32)
    haug_ref[:, :f] = h
    haug_ref[:, f:] = jnp.ones((N, 1), jnp.float32)
    f1 = jnp.dot(h, a_ref[:f, :], preferred_element_type=jnp.float32)
    f2 = jax.lax.dot_general(
        a_ref[f:, :], h, (((0,), (1,)), ((), ())),
        preferred_element_type=jnp.float32)
    f2_ref[...] = f2
    g_ref[...] = f2 * jnp.float32(ALPHA)
    m2 = jnp.max(f2)
    t = f1 + m2
    m = jnp.maximum(t, t * jnp.float32(ALPHA))
    c_ref[...] = f1 - m
    d_ref[...] = f1 * jnp.float32(ALPHA) - m
    cs_ref[...] = jnp.sum(h, axis=0, keepdims=True)


def _prep(x, w, a):
    f = w.shape[1]
    din = x.shape[1]
    return pl.pallas_call(
        functools.partial(_prep_kernel, f=f),
        grid=(1,),
        in_specs=[
            pl.BlockSpec((N, din), lambda i: (0, 0)),
            pl.BlockSpec((din, f), lambda i: (0, 0)),
            pl.BlockSpec((2 * f, 1), lambda i: (0, 0)),
        ],
        out_specs=[
            pl.BlockSpec((N, f + 1), lambda i: (0, 0)),
            pl.BlockSpec((N, 1), lambda i: (0, 0)),
            pl.BlockSpec((N, 1), lambda i: (0, 0)),
            pl.BlockSpec((1, N), lambda i: (0, 0)),
            pl.BlockSpec((1, N), lambda i: (0, 0)),
            pl.BlockSpec((1, f), lambda i: (0, 0)),
        ],
        out_shape=[
            jax.ShapeDtypeStruct((N, f + 1), jnp.float32),
            jax.ShapeDtypeStruct((N, 1), jnp.float32),
            jax.ShapeDtypeStruct((N, 1), jnp.float32),
            jax.ShapeDtypeStruct((1, N), jnp.float32),
            jax.ShapeDtypeStruct((1, N), jnp.float32),
            jax.ShapeDtypeStruct((1, f), jnp.float32),
        ],
    )(x, w, a)


def _softmax_block(adj_ref, haug_ref, c_ref, d_ref, f2_ref, g_ref, cs_ref, f):
    q = jnp.maximum(c_ref[...] + f2_ref[...], d_ref[...] + g_ref[...])
    pm = jnp.exp(q) * adj_ref[...]
    hpz = jnp.dot(pm, haug_ref[...], preferred_element_type=jnp.float32)
    s = hpz[:, f:]
    z = jnp.where(s == 0, jnp.float32(1.0), jnp.float32(0.0))
    r = jnp.float32(1.0) / (s + z * jnp.float32(N))
    hp = (hpz[:, :f] + z * cs_ref[...]) * r
    return pm, z, r, hp


def _layer1_kernel(adj_ref, haug_ref, c_ref, d_ref, f2_ref, g_ref, cs_ref,
                   out_ref, *, f):
    _, _, _, hp = _softmax_block(adj_ref, haug_ref, c_ref, d_ref, f2_ref,
                                 g_ref, cs_ref, f)
    out_ref[...] = jnp.maximum(hp, 0.0)


def _layer2_kernel(adj_ref, haug_ref, c_ref, d_ref, f2_ref, g_ref, cs_ref,
                   e2c_ref, b_ref, attn_ref, logits_ref, *, f):
    pm, z, r, hp = _softmax_block(adj_ref, haug_ref, c_ref, d_ref, f2_ref,
                                  g_ref, cs_ref, f)
    attn_ref[...] = (pm + z) * r
    logits_ref[...] = (
        jnp.dot(hp, e2c_ref[...], preferred_element_type=jnp.float32)
        + b_ref[...])


def _common_specs(f):
    return [
        pl.BlockSpec((BLK, N), lambda i: (i, 0)),
        pl.BlockSpec((N, f + 1), lambda i: (0, 0)),
        pl.BlockSpec((BLK, 1), lambda i: (i, 0)),
        pl.BlockSpec((BLK, 1), lambda i: (i, 0)),
        pl.BlockSpec((1, N), lambda i: (0, 0)),
        pl.BlockSpec((1, N), lambda i: (0, 0)),
        pl.BlockSpec((1, f), lambda i: (0, 0)),
    ]


def _layer1(adj, haug, c, d, f2, g, cs):
    f = haug.shape[1] - 1
    return pl.pallas_call(
        functools.partial(_layer1_kernel, f=f),
        grid=(N // BLK,),
        in_specs=_common_specs(f),
        out_specs=pl.BlockSpec((BLK, f), lambda i: (i, 0)),
        out_shape=jax.ShapeDtypeStruct((N, f), jnp.float32),
        compiler_params=pltpu.CompilerParams(
            dimension_semantics=("parallel",)),
    )(adj, haug, c, d, f2, g, cs)


def _layer2(adj, haug, c, d, f2, g, cs, e2c, b):
    f = haug.shape[1] - 1
    out = e2c.shape[1]
    return pl.pallas_call(
        functools.partial(_layer2_kernel, f=f),
        grid=(N // BLK,),
        in_specs=_common_specs(f) + [
            pl.BlockSpec((f, out), lambda i: (0, 0)),
            pl.BlockSpec((1, out), lambda i: (0, 0)),
        ],
        out_specs=[
            pl.BlockSpec((BLK, N), lambda i: (i, 0)),
            pl.BlockSpec((BLK, out), lambda i: (i, 0)),
        ],
        out_shape=[
            jax.ShapeDtypeStruct((N, N), jnp.float32),
            jax.ShapeDtypeStruct((N, out), jnp.float32),
        ],
        compiler_params=pltpu.CompilerParams(
            dimension_semantics=("parallel",)),
    )(adj, haug, c, d, f2, g, cs, e2c, b)


@jax.jit
def kernel(ft, adj, W1, a1, W2, a2, embd2class, bias):
    haug, c, d, f2, g, cs = _prep(ft, W1, a1)
    embd = _layer1(adj, haug, c, d, f2, g, cs)
    haug2, c2, d2, f22, g2, cs2 = _prep(embd, W2, a2)
    attention, logits = _layer2(adj, haug2, c2, d2, f22, g2, cs2,
                                embd2class, bias)
    return (logits, embd, attention)


# speedup vs baseline: 2.2434x; 1.0222x over previous
import functools

import jax
import jax.numpy as jnp
from jax.experimental import pallas as pl
from jax.experimental.pallas import tpu as pltpu

N = 4096
BLK = 512
ALPHA = 0.2
LOG2E = 1.4426950408889634


def _prep_kernel(x_ref, w_ref, a_ref, haug_ref, c_ref, d_ref, f2_ref,
                 g_ref, cs_ref, *, f):
    h = jnp.dot(x_ref[...], w_ref[...], preferred_element_type=jnp.float32)
    haug_ref[:, :f] = h
    haug_ref[:, f:] = jnp.ones((N, 1), jnp.float32)
    f1 = jnp.dot(h, a_ref[:f, :], preferred_element_type=jnp.float32)
    f2 = jax.lax.dot_general(
        a_ref[f:, :], h, (((0,), (1,)), ((), ())),
        preferred_element_type=jnp.float32)
    lg = jnp.float32(LOG2E)
    f2_ref[...] = f2 * lg
    g_ref[...] = f2 * jnp.float32(ALPHA * LOG2E)
    m2 = jnp.max(f2)
    t = f1 + m2
    m = jnp.maximum(t, t * jnp.float32(ALPHA))
    c_ref[...] = (f1 - m) * lg
    d_ref[...] = f1 * jnp.float32(ALPHA * LOG2E) - m * lg
    cs_ref[...] = jnp.sum(h, axis=0, keepdims=True)


def _prep(x, w, a):
    f = w.shape[1]
    din = x.shape[1]
    return pl.pallas_call(
        functools.partial(_prep_kernel, f=f),
        grid=(1,),
        in_specs=[
            pl.BlockSpec((N, din), lambda i: (0, 0)),
            pl.BlockSpec((din, f), lambda i: (0, 0)),
            pl.BlockSpec((2 * f, 1), lambda i: (0, 0)),
        ],
        out_specs=[
            pl.BlockSpec((N, f + 1), lambda i: (0, 0)),
            pl.BlockSpec((N, 1), lambda i: (0, 0)),
            pl.BlockSpec((N, 1), lambda i: (0, 0)),
            pl.BlockSpec((1, N), lambda i: (0, 0)),
            pl.BlockSpec((1, N), lambda i: (0, 0)),
            pl.BlockSpec((1, f), lambda i: (0, 0)),
        ],
        out_shape=[
            jax.ShapeDtypeStruct((N, f + 1), jnp.float32),
            jax.ShapeDtypeStruct((N, 1), jnp.float32),
            jax.ShapeDtypeStruct((N, 1), jnp.float32),
            jax.ShapeDtypeStruct((1, N), jnp.float32),
            jax.ShapeDtypeStruct((1, N), jnp.float32),
            jax.ShapeDtypeStruct((1, f), jnp.float32),
        ],
    )(x, w, a)


def _softmax_block(adj_ref, haug_ref, c_ref, d_ref, f2_ref, g_ref, cs_ref, f):
    q = jnp.maximum(c_ref[...] + f2_ref[...], d_ref[...] + g_ref[...])
    pm = jnp.exp2(q) * adj_ref[...]
    hpz = jnp.dot(pm, haug_ref[...], preferred_element_type=jnp.float32)
    s = hpz[:, f:]
    z = jnp.where(s == 0, jnp.float32(1.0), jnp.float32(0.0))
    r = jnp.float32(1.0) / (s + z * jnp.float32(N))
    hp = (hpz[:, :f] + z * cs_ref[...]) * r
    return pm, z, r, hp


def _layer1_kernel(adj_ref, haug_ref, c_ref, d_ref, f2_ref, g_ref, cs_ref,
                   out_ref, *, f):
    _, _, _, hp = _softmax_block(adj_ref, haug_ref, c_ref, d_ref, f2_ref,
                                 g_ref, cs_ref, f)
    out_ref[...] = jnp.maximum(hp, 0.0)


def _layer2_kernel(adj_ref, haug_ref, c_ref, d_ref, f2_ref, g_ref, cs_ref,
                   e2c_ref, b_ref, attn_ref, logits_ref, *, f):
    pm, z, r, hp = _softmax_block(adj_ref, haug_ref, c_ref, d_ref, f2_ref,
                                  g_ref, cs_ref, f)
    attn_ref[...] = (pm + z) * r
    logits_ref[...] = (
        jnp.dot(hp, e2c_ref[...], preferred_element_type=jnp.float32)
        + b_ref[...])


def _common_specs(f):
    return [
        pl.BlockSpec((BLK, N), lambda i: (i, 0)),
        pl.BlockSpec((N, f + 1), lambda i: (0, 0)),
        pl.BlockSpec((BLK, 1), lambda i: (i, 0)),
        pl.BlockSpec((BLK, 1), lambda i: (i, 0)),
        pl.BlockSpec((1, N), lambda i: (0, 0)),
        pl.BlockSpec((1, N), lambda i: (0, 0)),
        pl.BlockSpec((1, f), lambda i: (0, 0)),
    ]


def _layer1(adj, haug, c, d, f2, g, cs):
    f = haug.shape[1] - 1
    return pl.pallas_call(
        functools.partial(_layer1_kernel, f=f),
        grid=(N // BLK,),
        in_specs=_common_specs(f),
        out_specs=pl.BlockSpec((BLK, f), lambda i: (i, 0)),
        out_shape=jax.ShapeDtypeStruct((N, f), jnp.float32),
        compiler_params=pltpu.CompilerParams(
            dimension_semantics=("parallel",)),
    )(adj, haug, c, d, f2, g, cs)


def _layer2(adj, haug, c, d, f2, g, cs, e2c, b):
    f = haug.shape[1] - 1
    out = e2c.shape[1]
    return pl.pallas_call(
        functools.partial(_layer2_kernel, f=f),
        grid=(N // BLK,),
        in_specs=_common_specs(f) + [
            pl.BlockSpec((f, out), lambda i: (0, 0)),
            pl.BlockSpec((1, out), lambda i: (0, 0)),
        ],
        out_specs=[
            pl.BlockSpec((BLK, N), lambda i: (i, 0)),
            pl.BlockSpec((BLK, out), lambda i: (i, 0)),
        ],
        out_shape=[
            jax.ShapeDtypeStruct((N, N), jnp.float32),
            jax.ShapeDtypeStruct((N, out), jnp.float32),
        ],
        compiler_params=pltpu.CompilerParams(
            dimension_semantics=("parallel",)),
    )(adj, haug, c, d, f2, g, cs, e2c, b)


@jax.jit
def kernel(ft, adj, W1, a1, W2, a2, embd2class, bias):
    haug, c, d, f2, g, cs = _prep(ft, W1, a1)
    embd = _layer1(adj, haug, c, d, f2, g, cs)
    haug2, c2, d2, f22, g2, cs2 = _prep(embd, W2, a2)
    attention, logits = _layer2(adj, haug2, c2, d2, f22, g2, cs2,
                                embd2class, bias)
    return (logits, embd, attention)
